# Initial kernel scaffold; baseline (speedup 1.0000x reference)
import jax
import jax.numpy as jnp
from jax.experimental import pallas as pl


def kernel(nodes, edges, senders, receivers, globals_, params):
    raise NotImplementedError("write your pallas kernel here")


# speedup vs baseline: 5.6598x; 5.6598x over previous
import functools

import jax
import jax.numpy as jnp
from jax import lax
from jax.experimental import pallas as pl
from jax.experimental.pallas import tpu as pltpu
from jax.experimental.pallas import tpu_sc as plsc

DT = 0.01

_NC = 2
_NS = 16
_LN = 16


def _softplus(x):
    return jnp.maximum(x, 0.0) + jnp.log1p(jnp.exp(-jnp.abs(x)))


def _edge_encoder(edges, enc_edge):
    (W1, b1), (W2, b2) = enc_edge
    E = edges.shape[0]
    L = W2.shape[0]
    P = 128 // L
    assert E % (P * 1000) == 0
    rows = E // P

    e8 = edges.reshape(rows, P)
    R = jnp.zeros((P, P * L), jnp.float32)
    R = R.at[jnp.arange(P)[:, None], jnp.arange(P)[:, None] * L + jnp.arange(L)[None, :]].set(1.0)
    wt = jnp.tile(W1[0], P)[None, :]
    bt = jnp.tile(b1, P)[None, :]
    W2bd = jnp.kron(jnp.eye(P, dtype=jnp.float32), W2)
    b2t = jnp.tile(b2, P)[None, :]

    B = 4000
    grid = rows // B

    def body(e_ref, r_ref, wt_ref, bt_ref, w2_ref, b2_ref, o_ref):
        er = jnp.dot(e_ref[...], r_ref[...], preferred_element_type=jnp.float32)
        u = _softplus(er * wt_ref[...] + bt_ref[...])
        o_ref[...] = jnp.dot(u, w2_ref[...], preferred_element_type=jnp.float32) + b2_ref[...]

    full = lambda s: pl.BlockSpec(s, lambda i: (0, 0))
    out = pl.pallas_call(
        body,
        grid=(grid,),
        in_specs=[
            pl.BlockSpec((B, P), lambda i: (i, 0)),
            full((P, P * L)),
            full((1, P * L)),
            full((1, P * L)),
            full((P * L, P * L)),
            full((1, P * L)),
        ],
        out_specs=pl.BlockSpec((B, P * L), lambda i: (i, 0)),
        out_shape=jax.ShapeDtypeStruct((rows, P * L), jnp.float32),
    )(e8, R, wt, bt, W2bd, b2t)
    return out.reshape(E, L)


def _sc_dual_segment_sum(h_e, senders, receivers, n):
    E, L = h_e.shape
    assert L == _LN and E % 128 == 0 and n % _NS == 0
    rows = E // 128
    rpt = -(-rows // _NS)
    CH = 400
    assert n % CH == 0
    nch = n // CH
    cpt = -(-nch // _NS)

    mesh = plsc.VectorSubcoreMesh(core_axis_name="c", subcore_axis_name="s")

    @functools.partial(
        pl.kernel,
        out_type=(
            jax.ShapeDtypeStruct((n, L), jnp.float32),
            jax.ShapeDtypeStruct((n, L), jnp.float32),
        ),
        mesh=mesh,
        scratch_types=dict(
            hbuf=pltpu.VMEM((128, L), jnp.float32),
            ibuf=pltpu.VMEM((128,), jnp.int32),
            cbuf=pltpu.VMEM((CH, L), jnp.float32),
            accum=pltpu.VMEM_SHARED((n, L), jnp.float32),
        ),
        compiler_params=pltpu.CompilerParams(use_tc_tiling_on_sc=False),
    )
    def k(h_hbm, s_hbm, r_hbm, out_s, out_r, hbuf, ibuf, cbuf, accum):
        c = lax.axis_index("c")
        sid = lax.axis_index("s")
        r0 = sid * rpt
        r1 = jnp.minimum(rows, r0 + rpt)

        c0 = sid * cpt
        c1 = jnp.minimum(nch, c0 + cpt)

        def run_pass(idx_hbm, out_hbm):
            def zfill(i, carry):
                cbuf[i, :] = jnp.zeros((L,), jnp.float32)
                return carry

            lax.fori_loop(0, CH, zfill, 0)

            def zbody(j, carry):
                pltpu.sync_copy(cbuf, accum.at[pl.ds(j * CH, CH)])
                return carry

            lax.fori_loop(c0, c1, zbody, 0)
            plsc.subcore_barrier()

            def body(r, carry):
                pltpu.sync_copy(h_hbm.at[pl.ds(r * 128, 128)], hbuf)
                pltpu.sync_copy(idx_hbm.at[pl.ds(r * 128, 128)], ibuf)
                pltpu.sync_copy(hbuf, accum.at[ibuf], add=True)
                return carry

            lax.fori_loop(r0, r1, body, 0)
            plsc.subcore_barrier()

            def wbody(j, carry):
                pltpu.sync_copy(accum.at[pl.ds(j * CH, CH)], cbuf)
                pltpu.sync_copy(cbuf, out_hbm.at[pl.ds(j * CH, CH)])
                return carry

            lax.fori_loop(c0, c1, wbody, 0)

        @pl.when(c == 0)
        def _():
            run_pass(s_hbm, out_s)

        @pl.when(c == 1)
        def _():
            run_pass(r_hbm, out_r)

    return k(h_e, senders, receivers)


def _node_pipeline(nodes, sent, recv, globals_, params):
    N, F = nodes.shape
    L = sent.shape[1]
    (eW1, eb1), (eW2, eb2) = params["enc_node"]
    (pW1, pb1), (pW2, pb2) = params["proc_node"]
    (dW1, db1), (dW2, db2), (dW3, db3) = params["dec_node"]
    G = globals_.shape[0]
    pA = pW1[:L]
    pB = pW1[L : 2 * L]
    pC = pW1[2 * L : 3 * L]
    pD = pW1[3 * L :]
    g_row = globals_[None, :]

    B = 2000
    grid = N // B

    def body(n_ref, s_ref, r_ref, g_ref,
             ew1, eb1_, ew2, eb2_, pa, pb_, pc, pd, pb1_, pw2, pb2_,
             dw1, db1_, dw2, db2_, dw3, db3_, o_ref):
        x = n_ref[...]
        hn = _softplus(jnp.dot(x, ew1[...], preferred_element_type=jnp.float32) + eb1_[...])
        hn = jnp.dot(hn, ew2[...], preferred_element_type=jnp.float32) + eb2_[...]
        gv = jnp.dot(g_ref[...], pd[...], preferred_element_type=jnp.float32) + pb1_[...]
        t = (jnp.dot(hn, pa[...], preferred_element_type=jnp.float32)
             + jnp.dot(s_ref[...], pb_[...], preferred_element_type=jnp.float32)
             + jnp.dot(r_ref[...], pc[...], preferred_element_type=jnp.float32)
             + gv)
        h = jnp.dot(_softplus(t), pw2[...], preferred_element_type=jnp.float32) + pb2_[...]
        d = _softplus(jnp.dot(h, dw1[...], preferred_element_type=jnp.float32) + db1_[...])
        d = _softplus(jnp.dot(d, dw2[...], preferred_element_type=jnp.float32) + db2_[...])
        acc = jnp.dot(d, dw3[...], preferred_element_type=jnp.float32) + db3_[...]
        nv = x[:, F - 1 : F] + acc * DT
        np_ = x[:, 0:1] + nv * DT
        o_ref[...] = jnp.concatenate([np_, x[:, 2:F], nv, acc], axis=1)

    full = lambda a: pl.BlockSpec(a.shape, lambda i: tuple(0 for _ in a.shape))
    row2 = lambda v: v[None, :]
    weights = [eW1, row2(eb1), eW2, row2(eb2),
               pA, pB, pC, pD, row2(pb1), pW2, row2(pb2),
               dW1, row2(db1), dW2, row2(db2), dW3, row2(db3)]
    out = pl.pallas_call(
        body,
        grid=(grid,),
        in_specs=[
            pl.BlockSpec((B, F), lambda i: (i, 0)),
            pl.BlockSpec((B, L), lambda i: (i, 0)),
            pl.BlockSpec((B, L), lambda i: (i, 0)),
            full(g_row),
        ] + [full(w) for w in weights],
        out_specs=pl.BlockSpec((B, F + 1), lambda i: (i, 0)),
        out_shape=jax.ShapeDtypeStruct((N, F + 1), jnp.float32),
    )(nodes, sent, recv, g_row, *weights)
    return out


def kernel(nodes, edges, senders, receivers, globals_, params):
    N = nodes.shape[0]
    h_e = _edge_encoder(edges, params["enc_edge"])
    sent, recv = _sc_dual_segment_sum(h_e, senders, receivers, N)
    next_nodes = _node_pipeline(nodes, sent, recv, globals_, params)
    next_pos = next_nodes[:, 0]
    next_edges = (next_pos[1:] - next_pos[:-1]).reshape(-1, 1)
    new_globals = jnp.concatenate((globals_[:1] + 1.0, globals_[1:]))
    return next_nodes, next_edges, new_globals


# speedup vs baseline: 13.0497x; 2.3057x over previous
import functools

import jax
import jax.numpy as jnp
from jax import lax
from jax.experimental import pallas as pl
from jax.experimental.pallas import tpu as pltpu
from jax.experimental.pallas import tpu_sc as plsc

DT = 0.01

_NC = 2
_NS = 16
_LN = 16


def _softplus(x):
    return jnp.maximum(x, 0.0) + jnp.log1p(jnp.exp(-jnp.abs(x)))


def _edge_encoder(edges, enc_edge):
    (W1, b1), (W2, b2) = enc_edge
    E = edges.shape[0]
    L = W2.shape[0]
    P = 128 // L
    assert E % (P * 1000) == 0
    rows = E // P

    e8 = edges.reshape(rows, P)
    R = jnp.zeros((P, P * L), jnp.float32)
    R = R.at[jnp.arange(P)[:, None], jnp.arange(P)[:, None] * L + jnp.arange(L)[None, :]].set(1.0)
    wt = jnp.tile(W1[0], P)[None, :]
    bt = jnp.tile(b1, P)[None, :]
    W2bd = jnp.kron(jnp.eye(P, dtype=jnp.float32), W2)
    b2t = jnp.tile(b2, P)[None, :]

    B = 4000
    grid = rows // B

    def body(e_ref, r_ref, wt_ref, bt_ref, w2_ref, b2_ref, o_ref):
        er = jnp.dot(e_ref[...], r_ref[...], preferred_element_type=jnp.float32)
        u = _softplus(er * wt_ref[...] + bt_ref[...])
        o_ref[...] = jnp.dot(u, w2_ref[...], preferred_element_type=jnp.float32) + b2_ref[...]

    full = lambda s: pl.BlockSpec(s, lambda i: (0, 0))
    out = pl.pallas_call(
        body,
        grid=(grid,),
        in_specs=[
            pl.BlockSpec((B, P), lambda i: (i, 0)),
            full((P, P * L)),
            full((1, P * L)),
            full((1, P * L)),
            full((P * L, P * L)),
            full((1, P * L)),
        ],
        out_specs=pl.BlockSpec((B, P * L), lambda i: (i, 0)),
        out_shape=jax.ShapeDtypeStruct((rows, P * L), jnp.float32),
    )(e8, R, wt, bt, W2bd, b2t)
    return out.reshape(E, L)


def _sc_dual_segment_sum(h_e, senders, receivers, n):
    E, L = h_e.shape
    CHB = 4
    CE = CHB * 128
    assert L == _LN and E % CE == 0 and n % _NS == 0
    chunks = E // CE
    kpt = -(-chunks // _NS)
    CH = 200
    assert n % CH == 0
    nch = n // CH
    cpt = -(-nch // _NS)

    s2d = senders.reshape(E // 128, 128)
    r2d = receivers.reshape(E // 128, 128)

    mesh = plsc.VectorSubcoreMesh(core_axis_name="c", subcore_axis_name="s")

    @functools.partial(
        pl.kernel,
        out_type=(
            jax.ShapeDtypeStruct((n, L), jnp.float32),
            jax.ShapeDtypeStruct((n, L), jnp.float32),
        ),
        mesh=mesh,
        scratch_types=dict(
            hbuf0=pltpu.VMEM((CE, L), jnp.float32),
            hbuf1=pltpu.VMEM((CE, L), jnp.float32),
            ibuf0=pltpu.VMEM((CHB, 128), jnp.int32),
            ibuf1=pltpu.VMEM((CHB, 128), jnp.int32),
            cbuf=pltpu.VMEM((CH, L), jnp.float32),
            accum=pltpu.VMEM_SHARED((n, L), jnp.float32),
            lsem0=pltpu.SemaphoreType.DMA,
            lsem1=pltpu.SemaphoreType.DMA,
            ssem=pltpu.SemaphoreType.DMA,
        ),
        compiler_params=pltpu.CompilerParams(use_tc_tiling_on_sc=False),
    )
    def k(h_hbm, s_hbm, r_hbm, out_s, out_r,
          hbuf0, hbuf1, ibuf0, ibuf1, cbuf, accum, lsem0, lsem1, ssem):
        c = lax.axis_index("c")
        sid = lax.axis_index("s")
        g0 = sid * kpt
        g1 = jnp.minimum(chunks, g0 + kpt)
        n_my = jnp.maximum(0, g1 - g0)

        c0 = sid * cpt
        c1 = jnp.minimum(nch, c0 + cpt)

        bufs = ((hbuf0, ibuf0, lsem0), (hbuf1, ibuf1, lsem1))

        def run_pass(idx_hbm, out_hbm):
            def zfill(i, carry):
                cbuf[i, :] = jnp.zeros((L,), jnp.float32)
                return carry

            lax.fori_loop(0, CH, zfill, 0)

            def zbody(j, carry):
                pltpu.sync_copy(cbuf, accum.at[pl.ds(j * CH, CH)])
                return carry

            lax.fori_loop(c0, c1, zbody, 0)
            plsc.subcore_barrier()

            def loads(i, b):
                hb, ib, ls = bufs[b]
                g = g0 + i
                hd = pltpu.async_copy(h_hbm.at[pl.ds(g * CE, CE)], hb, ls)
                idd = pltpu.async_copy(idx_hbm.at[pl.ds(g * CHB, CHB)], ib, ls)
                return hd, idd

            @pl.when(n_my > 0)
            def _():
                loads(0, 0)

            def process(i, b):
                hb, ib, ls = bufs[b]

                @pl.when(i < n_my)
                def _():
                    g = g0 + i
                    pltpu.make_async_copy(h_hbm.at[pl.ds(g * CE, CE)], hb, ls).wait()
                    pltpu.make_async_copy(idx_hbm.at[pl.ds(g * CHB, CHB)], ib, ls).wait()

                    @pl.when(i + 1 < n_my)
                    def _():
                        loads(i + 1, 1 - b)

                    descs = [
                        pltpu.async_copy(
                            hb.at[pl.ds(j * 128, 128)], accum.at[ib.at[j]],
                            ssem, add=True)
                        for j in range(CHB)
                    ]
                    for d in descs:
                        d.wait()

            def pairbody(q, carry):
                process(2 * q, 0)
                process(2 * q + 1, 1)
                return carry

            lax.fori_loop(0, (kpt + 1) // 2, pairbody, 0)
            plsc.subcore_barrier()

            def wbody(j, carry):
                pltpu.sync_copy(accum.at[pl.ds(j * CH, CH)], cbuf)
                pltpu.sync_copy(cbuf, out_hbm.at[pl.ds(j * CH, CH)])
                return carry

            lax.fori_loop(c0, c1, wbody, 0)

        @pl.when(c == 0)
        def _():
            run_pass(s_hbm, out_s)

        @pl.when(c == 1)
        def _():
            run_pass(r_hbm, out_r)

    return k(h_e, s2d, r2d)


def _node_pipeline(nodes, sent, recv, globals_, params):
    N, F = nodes.shape
    L = sent.shape[1]
    (eW1, eb1), (eW2, eb2) = params["enc_node"]
    (pW1, pb1), (pW2, pb2) = params["proc_node"]
    (dW1, db1), (dW2, db2), (dW3, db3) = params["dec_node"]
    G = globals_.shape[0]
    pA = pW1[:L]
    pB = pW1[L : 2 * L]
    pC = pW1[2 * L : 3 * L]
    pD = pW1[3 * L :]
    g_row = globals_[None, :]

    B = 2000
    grid = N // B

    def body(n_ref, s_ref, r_ref, g_ref,
             ew1, eb1_, ew2, eb2_, pa, pb_, pc, pd, pb1_, pw2, pb2_,
             dw1, db1_, dw2, db2_, dw3, db3_, o_ref):
        x = n_ref[...]
        hn = _softplus(jnp.dot(x, ew1[...], preferred_element_type=jnp.float32) + eb1_[...])
        hn = jnp.dot(hn, ew2[...], preferred_element_type=jnp.float32) + eb2_[...]
        gv = jnp.dot(g_ref[...], pd[...], preferred_element_type=jnp.float32) + pb1_[...]
        t = (jnp.dot(hn, pa[...], preferred_element_type=jnp.float32)
             + jnp.dot(s_ref[...], pb_[...], preferred_element_type=jnp.float32)
             + jnp.dot(r_ref[...], pc[...], preferred_element_type=jnp.float32)
             + gv)
        h = jnp.dot(_softplus(t), pw2[...], preferred_element_type=jnp.float32) + pb2_[...]
        d = _softplus(jnp.dot(h, dw1[...], preferred_element_type=jnp.float32) + db1_[...])
        d = _softplus(jnp.dot(d, dw2[...], preferred_element_type=jnp.float32) + db2_[...])
        acc = jnp.dot(d, dw3[...], preferred_element_type=jnp.float32) + db3_[...]
        nv = x[:, F - 1 : F] + acc * DT
        np_ = x[:, 0:1] + nv * DT
        o_ref[...] = jnp.concatenate([np_, x[:, 2:F], nv, acc], axis=1)

    full = lambda a: pl.BlockSpec(a.shape, lambda i: tuple(0 for _ in a.shape))
    row2 = lambda v: v[None, :]
    weights = [eW1, row2(eb1), eW2, row2(eb2),
               pA, pB, pC, pD, row2(pb1), pW2, row2(pb2),
               dW1, row2(db1), dW2, row2(db2), dW3, row2(db3)]
    out = pl.pallas_call(
        body,
        grid=(grid,),
        in_specs=[
            pl.BlockSpec((B, F), lambda i: (i, 0)),
            pl.BlockSpec((B, L), lambda i: (i, 0)),
            pl.BlockSpec((B, L), lambda i: (i, 0)),
            full(g_row),
        ] + [full(w) for w in weights],
        out_specs=pl.BlockSpec((B, F + 1), lambda i: (i, 0)),
        out_shape=jax.ShapeDtypeStruct((N, F + 1), jnp.float32),
    )(nodes, sent, recv, g_row, *weights)
    return out


def kernel(nodes, edges, senders, receivers, globals_, params):
    N = nodes.shape[0]
    h_e = _edge_encoder(edges, params["enc_edge"])
    sent, recv = _sc_dual_segment_sum(h_e, senders, receivers, N)
    next_nodes = _node_pipeline(nodes, sent, recv, globals_, params)
    next_pos = next_nodes[:, 0]
    next_edges = (next_pos[1:] - next_pos[:-1]).reshape(-1, 1)
    new_globals = jnp.concatenate((globals_[:1] + 1.0, globals_[1:]))
    return next_nodes, next_edges, new_globals


# speedup vs baseline: 14.4874x; 1.1102x over previous
import functools

import jax
import jax.numpy as jnp
from jax import lax
from jax.experimental import pallas as pl
from jax.experimental.pallas import tpu as pltpu
from jax.experimental.pallas import tpu_sc as plsc

DT = 0.01

_NC = 2
_NS = 16
_LN = 16


def _softplus(x):
    return jnp.maximum(x, 0.0) + jnp.log1p(jnp.exp(-jnp.abs(x)))


def _edge_encoder(edges, enc_edge):
    (W1, b1), (W2, b2) = enc_edge
    E = edges.shape[0]
    L = W2.shape[0]
    P = 128 // L
    assert E % (P * 1000) == 0
    rows = E // P

    e8 = edges.reshape(rows, P)
    R = jnp.zeros((P, P * L), jnp.float32)
    R = R.at[jnp.arange(P)[:, None], jnp.arange(P)[:, None] * L + jnp.arange(L)[None, :]].set(1.0)
    wt = jnp.tile(W1[0], P)[None, :]
    bt = jnp.tile(b1, P)[None, :]
    W2bd = jnp.kron(jnp.eye(P, dtype=jnp.float32), W2)
    b2t = jnp.tile(b2, P)[None, :]

    B = 4000
    grid = rows // B

    def body(e_ref, r_ref, wt_ref, bt_ref, w2_ref, b2_ref, o_ref):
        er = jnp.dot(e_ref[...], r_ref[...], preferred_element_type=jnp.float32)
        u = _softplus(er * wt_ref[...] + bt_ref[...])
        o_ref[...] = jnp.dot(u, w2_ref[...], preferred_element_type=jnp.float32) + b2_ref[...]

    full = lambda s: pl.BlockSpec(s, lambda i: (0, 0))
    out = pl.pallas_call(
        body,
        grid=(grid,),
        in_specs=[
            pl.BlockSpec((B, P), lambda i: (i, 0)),
            full((P, P * L)),
            full((1, P * L)),
            full((1, P * L)),
            full((P * L, P * L)),
            full((1, P * L)),
        ],
        out_specs=pl.BlockSpec((B, P * L), lambda i: (i, 0)),
        out_shape=jax.ShapeDtypeStruct((rows, P * L), jnp.float32),
    )(e8, R, wt, bt, W2bd, b2t)
    return out.reshape(E, L)


def _sc_dual_segment_sum(h_e, senders, receivers, n):
    E, L = h_e.shape
    CHB = 4
    CE = CHB * 128
    assert L == _LN and E % CE == 0 and n % _NS == 0
    chunks = E // CE
    kpt = -(-chunks // _NS)
    CH = 200
    assert n % CH == 0
    nch = n // CH
    cpt = -(-nch // _NS)

    s2d = senders.reshape(E // 128, 128)
    r2d = receivers.reshape(E // 128, 128)

    mesh = plsc.VectorSubcoreMesh(core_axis_name="c", subcore_axis_name="s")

    @functools.partial(
        pl.kernel,
        out_type=(
            jax.ShapeDtypeStruct((n, L), jnp.float32),
            jax.ShapeDtypeStruct((n, L), jnp.float32),
        ),
        mesh=mesh,
        scratch_types=dict(
            hbuf0=pltpu.VMEM((CE, L), jnp.float32),
            hbuf1=pltpu.VMEM((CE, L), jnp.float32),
            ibuf0=pltpu.VMEM((CHB, 128), jnp.int32),
            ibuf1=pltpu.VMEM((CHB, 128), jnp.int32),
            cbuf=pltpu.VMEM((CH, L), jnp.float32),
            accum=pltpu.VMEM_SHARED((n, L), jnp.float32),
            lsem0=pltpu.SemaphoreType.DMA,
            lsem1=pltpu.SemaphoreType.DMA,
            ssem0=pltpu.SemaphoreType.DMA,
            ssem1=pltpu.SemaphoreType.DMA,
        ),
        compiler_params=pltpu.CompilerParams(use_tc_tiling_on_sc=False),
    )
    def k(h_hbm, s_hbm, r_hbm, out_s, out_r,
          hbuf0, hbuf1, ibuf0, ibuf1, cbuf, accum, lsem0, lsem1, ssem0, ssem1):
        c = lax.axis_index("c")
        sid = lax.axis_index("s")
        g0 = sid * kpt
        g1 = jnp.minimum(chunks, g0 + kpt)
        n_my = jnp.maximum(0, g1 - g0)

        c0 = sid * cpt
        c1 = jnp.minimum(nch, c0 + cpt)

        bufs = ((hbuf0, ibuf0, lsem0, ssem0), (hbuf1, ibuf1, lsem1, ssem1))

        def run_pass(idx_hbm, out_hbm):
            def zfill(i, carry):
                cbuf[i, :] = jnp.zeros((L,), jnp.float32)
                return carry

            lax.fori_loop(0, CH, zfill, 0)

            def zbody(j, carry):
                pltpu.sync_copy(cbuf, accum.at[pl.ds(j * CH, CH)])
                return carry

            lax.fori_loop(c0, c1, zbody, 0)
            plsc.subcore_barrier()

            def loads(i, b):
                hb, ib, ls, _ = bufs[b]
                g = g0 + i
                pltpu.async_copy(h_hbm.at[pl.ds(g * CE, CE)], hb, ls)
                pltpu.async_copy(idx_hbm.at[pl.ds(g * CHB, CHB)], ib, ls)

            def drain_scatters(b):
                hb, ib, _, ss = bufs[b]
                for j in range(CHB):
                    pltpu.make_async_copy(
                        hb.at[pl.ds(j * 128, 128)], accum.at[ib.at[j]], ss
                    ).wait()

            @pl.when(n_my > 0)
            def _():
                loads(0, 0)

            def process(i, b):
                hb, ib, ls, ss = bufs[b]

                @pl.when(i < n_my)
                def _():
                    g = g0 + i
                    pltpu.make_async_copy(h_hbm.at[pl.ds(g * CE, CE)], hb, ls).wait()
                    pltpu.make_async_copy(idx_hbm.at[pl.ds(g * CHB, CHB)], ib, ls).wait()

                    @pl.when(i + 1 < n_my)
                    def _():
                        @pl.when(i >= 1)
                        def _():
                            drain_scatters(1 - b)

                        loads(i + 1, 1 - b)

                    for j in range(CHB):
                        pltpu.async_copy(
                            hb.at[pl.ds(j * 128, 128)], accum.at[ib.at[j]],
                            ss, add=True)

            def pairbody(q, carry):
                process(2 * q, 0)
                process(2 * q + 1, 1)
                return carry

            lax.fori_loop(0, (kpt + 1) // 2, pairbody, 0)
            @pl.when(n_my >= 1)
            def _():
                drain_scatters((0))

            @pl.when(n_my >= 2)
            def _():
                drain_scatters((1))

            plsc.subcore_barrier()

            def wbody(j, carry):
                pltpu.sync_copy(accum.at[pl.ds(j * CH, CH)], cbuf)
                pltpu.sync_copy(cbuf, out_hbm.at[pl.ds(j * CH, CH)])
                return carry

            lax.fori_loop(c0, c1, wbody, 0)

        @pl.when(c == 0)
        def _():
            run_pass(s_hbm, out_s)

        @pl.when(c == 1)
        def _():
            run_pass(r_hbm, out_r)

    return k(h_e, s2d, r2d)


def _node_pipeline(nodes, sentP, recvP, globals_, params):
    N, F = nodes.shape
    L = 16
    P8 = 128 // L
    (eW1, eb1), (eW2, eb2) = params["enc_node"]
    (pW1, pb1), (pW2, pb2) = params["proc_node"]
    (dW1, db1), (dW2, db2), (dW3, db3) = params["dec_node"]
    pA = pW1[:L]
    pB = pW1[L : 2 * L]
    pC = pW1[2 * L : 3 * L]
    pD = pW1[3 * L :]
    g_row = globals_[None, :]
    eyeP = jnp.eye(P8, dtype=jnp.float32)
    BDB = jnp.kron(eyeP, pB)
    BDC = jnp.kron(eyeP, pC)

    B = 2048
    grid = -(-N // B)

    def body(n_ref, s_ref, r_ref, g_ref,
             ew1, eb1_, ew2, eb2_, pa, bdb, bdc, pd, pb1_, pw2, pb2_,
             dw1, db1_, dw2, db2_, dw3t, db3_, o_ref):
        x = n_ref[...]
        xT = jnp.transpose(x)
        hn = _softplus(jnp.dot(x, ew1[...], preferred_element_type=jnp.float32) + eb1_[...])
        hn = jnp.dot(hn, ew2[...], preferred_element_type=jnp.float32) + eb2_[...]
        cP = (jnp.dot(s_ref[...], bdb[...], preferred_element_type=jnp.float32)
              + jnp.dot(r_ref[...], bdc[...], preferred_element_type=jnp.float32))
        c = jnp.stack([cP[:, L * e : L * (e + 1)] for e in range(P8)], axis=1)
        c = c.reshape(B, L)
        gv = jnp.dot(g_ref[...], pd[...], preferred_element_type=jnp.float32) + pb1_[...]
        t = jnp.dot(hn, pa[...], preferred_element_type=jnp.float32) + c + gv
        h = jnp.dot(_softplus(t), pw2[...], preferred_element_type=jnp.float32) + pb2_[...]
        d = _softplus(jnp.dot(h, dw1[...], preferred_element_type=jnp.float32) + db1_[...])
        d = _softplus(jnp.dot(d, dw2[...], preferred_element_type=jnp.float32) + db2_[...])
        accT = (jnp.dot(dw3t[...], jnp.transpose(d),
                        preferred_element_type=jnp.float32)
                + db3_[...])
        nvT = xT[F - 1 : F] + accT * DT
        npT = xT[0:1] + nvT * DT
        o_ref[...] = jnp.concatenate([npT, xT[2:F], nvT, accT], axis=0)

    full = lambda a: pl.BlockSpec(a.shape, lambda i: tuple(0 for _ in a.shape))
    row2 = lambda v: v[None, :]
    weights = [eW1, row2(eb1), eW2, row2(eb2),
               pA, BDB, BDC, pD, row2(pb1), pW2, row2(pb2),
               dW1, row2(db1), dW2, row2(db2), dW3.T, db3.reshape(1, 1)]
    out = pl.pallas_call(
        body,
        grid=(grid,),
        in_specs=[
            pl.BlockSpec((B, F), lambda i: (i, 0)),
            pl.BlockSpec((B // P8, 128), lambda i: (i, 0)),
            pl.BlockSpec((B // P8, 128), lambda i: (i, 0)),
            full(g_row),
        ] + [full(w) for w in weights],
        out_specs=pl.BlockSpec((F + 1, B), lambda i: (0, i)),
        out_shape=jax.ShapeDtypeStruct((F + 1, N), jnp.float32),
    )(nodes, sentP, recvP, g_row, *weights)
    return out


def kernel(nodes, edges, senders, receivers, globals_, params):
    N = nodes.shape[0]
    h_e = _edge_encoder(edges, params["enc_edge"])
    sent, recv = _sc_dual_segment_sum(h_e, senders, receivers, N)
    outT = _node_pipeline(
        nodes, sent.reshape(N // 8, 128), recv.reshape(N // 8, 128),
        globals_, params)
    next_nodes = outT.T
    next_pos = outT[0]
    next_edges = (next_pos[1:] - next_pos[:-1]).reshape(-1, 1)
    new_globals = jnp.concatenate((globals_[:1] + 1.0, globals_[1:]))
    return next_nodes, next_edges, new_globals


# speedup vs baseline: 14.5170x; 1.0020x over previous
import functools

import jax
import jax.numpy as jnp
from jax import lax
from jax.experimental import pallas as pl
from jax.experimental.pallas import tpu as pltpu
from jax.experimental.pallas import tpu_sc as plsc

DT = 0.01

_NC = 2
_NS = 16
_LN = 16


def _softplus(x):
    return jnp.maximum(x, 0.0) + jnp.log1p(jnp.exp(-jnp.abs(x)))


def _edge_encoder(edges, enc_edge):
    (W1, b1), (W2, b2) = enc_edge
    E = edges.shape[0]
    L = W2.shape[0]
    P = 128 // L
    assert E % (P * 1000) == 0
    rows = E // P

    e8 = edges.reshape(rows, P)
    R = jnp.zeros((P, P * L), jnp.float32)
    R = R.at[jnp.arange(P)[:, None], jnp.arange(P)[:, None] * L + jnp.arange(L)[None, :]].set(1.0)
    wt = jnp.tile(W1[0], P)[None, :]
    bt = jnp.tile(b1, P)[None, :]
    W2bd = jnp.kron(jnp.eye(P, dtype=jnp.float32), W2)
    b2t = jnp.tile(b2, P)[None, :]

    B = 4000
    grid = rows // B

    def body(e_ref, r_ref, wt_ref, bt_ref, w2_ref, b2_ref, o_ref):
        er = jnp.dot(e_ref[...], r_ref[...], preferred_element_type=jnp.float32)
        u = _softplus(er * wt_ref[...] + bt_ref[...])
        o_ref[...] = jnp.dot(u, w2_ref[...], preferred_element_type=jnp.float32) + b2_ref[...]

    full = lambda s: pl.BlockSpec(s, lambda i: (0, 0))
    out = pl.pallas_call(
        body,
        grid=(grid,),
        in_specs=[
            pl.BlockSpec((B, P), lambda i: (i, 0)),
            full((P, P * L)),
            full((1, P * L)),
            full((1, P * L)),
            full((P * L, P * L)),
            full((1, P * L)),
        ],
        out_specs=pl.BlockSpec((B, P * L), lambda i: (i, 0)),
        out_shape=jax.ShapeDtypeStruct((rows, P * L), jnp.float32),
    )(e8, R, wt, bt, W2bd, b2t)
    return out.reshape(E, L)


def _sc_dual_segment_sum(h_e, senders, receivers, n):
    E, L = h_e.shape
    CHB = 4
    CE = CHB * 128
    assert L == _LN and E % CE == 0 and n % _NS == 0
    chunks = E // CE
    kpt = -(-chunks // _NS)
    CH = 200
    assert n % CH == 0
    nch = n // CH
    cpt = -(-nch // _NS)


    mesh = plsc.VectorSubcoreMesh(core_axis_name="c", subcore_axis_name="s")

    @functools.partial(
        pl.kernel,
        out_type=(
            jax.ShapeDtypeStruct((n, L), jnp.float32),
            jax.ShapeDtypeStruct((n, L), jnp.float32),
        ),
        mesh=mesh,
        scratch_types=dict(
            hbuf0=pltpu.VMEM((CE, L), jnp.float32),
            hbuf1=pltpu.VMEM((CE, L), jnp.float32),
            ibuf0=pltpu.VMEM((CE,), jnp.int32),
            ibuf1=pltpu.VMEM((CE,), jnp.int32),
            cbuf=pltpu.VMEM((CH, L), jnp.float32),
            accum=pltpu.VMEM_SHARED((n, L), jnp.float32),
            lsem0=pltpu.SemaphoreType.DMA,
            lsem1=pltpu.SemaphoreType.DMA,
            ssem0=pltpu.SemaphoreType.DMA,
            ssem1=pltpu.SemaphoreType.DMA,
        ),
        compiler_params=pltpu.CompilerParams(use_tc_tiling_on_sc=False),
    )
    def k(h_hbm, s_hbm, r_hbm, out_s, out_r,
          hbuf0, hbuf1, ibuf0, ibuf1, cbuf, accum, lsem0, lsem1, ssem0, ssem1):
        c = lax.axis_index("c")
        sid = lax.axis_index("s")
        g0 = sid * kpt
        g1 = jnp.minimum(chunks, g0 + kpt)
        n_my = jnp.maximum(0, g1 - g0)

        c0 = sid * cpt
        c1 = jnp.minimum(nch, c0 + cpt)

        bufs = ((hbuf0, ibuf0, lsem0, ssem0), (hbuf1, ibuf1, lsem1, ssem1))

        def run_pass(idx_hbm, out_hbm):
            def zfill(i, carry):
                cbuf[i, :] = jnp.zeros((L,), jnp.float32)
                return carry

            lax.fori_loop(0, CH, zfill, 0)

            def zbody(j, carry):
                pltpu.sync_copy(cbuf, accum.at[pl.ds(j * CH, CH)])
                return carry

            lax.fori_loop(c0, c1, zbody, 0)
            plsc.subcore_barrier()

            def loads(i, b):
                hb, ib, ls, _ = bufs[b]
                g = g0 + i
                pltpu.async_copy(h_hbm.at[pl.ds(g * CE, CE)], hb, ls)
                pltpu.async_copy(idx_hbm.at[pl.ds(g * CE, CE)], ib, ls)

            def drain_scatters(b):
                hb, ib, _, ss = bufs[b]
                pltpu.make_async_copy(hb, accum.at[ib], ss).wait()

            @pl.when(n_my > 0)
            def _():
                loads(0, 0)

            def process(i, b):
                hb, ib, ls, ss = bufs[b]

                @pl.when(i < n_my)
                def _():
                    g = g0 + i
                    pltpu.make_async_copy(h_hbm.at[pl.ds(g * CE, CE)], hb, ls).wait()
                    pltpu.make_async_copy(idx_hbm.at[pl.ds(g * CE, CE)], ib, ls).wait()

                    @pl.when(i + 1 < n_my)
                    def _():
                        @pl.when(i >= 1)
                        def _():
                            drain_scatters(1 - b)

                        loads(i + 1, 1 - b)

                    pltpu.async_copy(hb, accum.at[ib], ss, add=True)

            def pairbody(q, carry):
                process(2 * q, 0)
                process(2 * q + 1, 1)
                return carry

            lax.fori_loop(0, (kpt + 1) // 2, pairbody, 0)
            @pl.when(n_my >= 1)
            def _():
                drain_scatters((0))

            @pl.when(n_my >= 2)
            def _():
                drain_scatters((1))

            plsc.subcore_barrier()

            def wbody(j, carry):
                pltpu.sync_copy(accum.at[pl.ds(j * CH, CH)], cbuf)
                pltpu.sync_copy(cbuf, out_hbm.at[pl.ds(j * CH, CH)])
                return carry

            lax.fori_loop(c0, c1, wbody, 0)

        @pl.when(c == 0)
        def _():
            run_pass(s_hbm, out_s)

        @pl.when(c == 1)
        def _():
            run_pass(r_hbm, out_r)

    return k(h_e, senders, receivers)


def _node_pipeline(nodes, sentP, recvP, globals_, params):
    N, F = nodes.shape
    L = 16
    P8 = 128 // L
    (eW1, eb1), (eW2, eb2) = params["enc_node"]
    (pW1, pb1), (pW2, pb2) = params["proc_node"]
    (dW1, db1), (dW2, db2), (dW3, db3) = params["dec_node"]
    pA = pW1[:L]
    pB = pW1[L : 2 * L]
    pC = pW1[2 * L : 3 * L]
    pD = pW1[3 * L :]
    g_row = globals_[None, :]
    eyeP = jnp.eye(P8, dtype=jnp.float32)
    BDB = jnp.kron(eyeP, pB)
    BDC = jnp.kron(eyeP, pC)

    B = 2048
    grid = -(-N // B)

    def body(n_ref, s_ref, r_ref, g_ref,
             ew1, eb1_, ew2, eb2_, pa, bdb, bdc, pd, pb1_, pw2, pb2_,
             dw1, db1_, dw2, db2_, dw3t, db3_, o_ref):
        x = n_ref[...]
        xT = jnp.transpose(x)
        hn = _softplus(jnp.dot(x, ew1[...], preferred_element_type=jnp.float32) + eb1_[...])
        hn = jnp.dot(hn, ew2[...], preferred_element_type=jnp.float32) + eb2_[...]
        cP = (jnp.dot(s_ref[...], bdb[...], preferred_element_type=jnp.float32)
              + jnp.dot(r_ref[...], bdc[...], preferred_element_type=jnp.float32))
        c = jnp.stack([cP[:, L * e : L * (e + 1)] for e in range(P8)], axis=1)
        c = c.reshape(B, L)
        gv = jnp.dot(g_ref[...], pd[...], preferred_element_type=jnp.float32) + pb1_[...]
        t = jnp.dot(hn, pa[...], preferred_element_type=jnp.float32) + c + gv
        h = jnp.dot(_softplus(t), pw2[...], preferred_element_type=jnp.float32) + pb2_[...]
        d = _softplus(jnp.dot(h, dw1[...], preferred_element_type=jnp.float32) + db1_[...])
        d = _softplus(jnp.dot(d, dw2[...], preferred_element_type=jnp.float32) + db2_[...])
        accT = (jnp.dot(dw3t[...], jnp.transpose(d),
                        preferred_element_type=jnp.float32)
                + db3_[...])
        nvT = xT[F - 1 : F] + accT * DT
        npT = xT[0:1] + nvT * DT
        o_ref[...] = jnp.concatenate([npT, xT[2:F], nvT, accT], axis=0)

    full = lambda a: pl.BlockSpec(a.shape, lambda i: tuple(0 for _ in a.shape))
    row2 = lambda v: v[None, :]
    weights = [eW1, row2(eb1), eW2, row2(eb2),
               pA, BDB, BDC, pD, row2(pb1), pW2, row2(pb2),
               dW1, row2(db1), dW2, row2(db2), dW3.T, db3.reshape(1, 1)]
    out = pl.pallas_call(
        body,
        grid=(grid,),
        in_specs=[
            pl.BlockSpec((B, F), lambda i: (i, 0)),
            pl.BlockSpec((B // P8, 128), lambda i: (i, 0)),
            pl.BlockSpec((B // P8, 128), lambda i: (i, 0)),
            full(g_row),
        ] + [full(w) for w in weights],
        out_specs=pl.BlockSpec((F + 1, B), lambda i: (0, i)),
        out_shape=jax.ShapeDtypeStruct((F + 1, N), jnp.float32),
    )(nodes, sentP, recvP, g_row, *weights)
    return out


def kernel(nodes, edges, senders, receivers, globals_, params):
    N = nodes.shape[0]
    h_e = _edge_encoder(edges, params["enc_edge"])
    sent, recv = _sc_dual_segment_sum(h_e, senders, receivers, N)
    outT = _node_pipeline(
        nodes, sent.reshape(N // 8, 128), recv.reshape(N // 8, 128),
        globals_, params)
    next_nodes = outT.T
    next_pos = outT[0]
    next_edges = (next_pos[1:] - next_pos[:-1]).reshape(-1, 1)
    new_globals = jnp.concatenate((globals_[:1] + 1.0, globals_[1:]))
    return next_nodes, next_edges, new_globals


# speedup vs baseline: 16.3093x; 1.1235x over previous
import functools

import jax
import jax.numpy as jnp
from jax import lax
from jax.experimental import pallas as pl
from jax.experimental.pallas import tpu as pltpu
from jax.experimental.pallas import tpu_sc as plsc

DT = 0.01

_NC = 2
_NS = 16
_LN = 16


def _softplus(x):
    return jnp.maximum(x, 0.0) + jnp.log1p(jnp.exp(-jnp.abs(x)))


def _edge_encoder(edges, enc_edge):
    (W1, b1), (W2, b2) = enc_edge
    E = edges.shape[0]
    L = W2.shape[0]
    W = 128 * L

    e128 = edges.reshape(E // 128, 128)
    c = jnp.linalg.solve(W2, b2)
    R2 = jnp.zeros((128, W), jnp.float32)
    R2 = R2.at[jnp.arange(128)[:, None],
               jnp.arange(128)[:, None] * L + jnp.arange(L)[None, :]].set(1.0)
    wt = jnp.tile(W1[0], 128)[None, :]
    bt = jnp.tile(b1, 128)[None, :]
    ct = jnp.tile(c, 128)[None, :]

    B128 = 200
    assert (E // 128) % B128 == 0
    grid = (E // 128) // B128

    def body(e_ref, r_ref, wt_ref, bt_ref, ct_ref, o_ref):
        er = jnp.dot(e_ref[...], r_ref[...], preferred_element_type=jnp.float32)
        o_ref[...] = _softplus(er * wt_ref[...] + bt_ref[...]) + ct_ref[...]

    full = lambda s: pl.BlockSpec(s, lambda i: (0, 0))
    out = pl.pallas_call(
        body,
        grid=(grid,),
        in_specs=[
            pl.BlockSpec((B128, 128), lambda i: (i, 0)),
            full((128, W)),
            full((1, W)),
            full((1, W)),
            full((1, W)),
        ],
        out_specs=pl.BlockSpec((B128, W), lambda i: (i, 0)),
        out_shape=jax.ShapeDtypeStruct((E // 128, W), jnp.float32),
    )(e128, R2, wt, bt, ct)
    return out.reshape(E, L)


def _sc_dual_segment_sum(h_e, senders, receivers, n):
    E, L = h_e.shape
    CHB = 4
    CE = CHB * 128
    assert L == _LN and E % CE == 0 and n % _NS == 0
    chunks = E // CE
    kpt = -(-chunks // _NS)
    CH = 200
    assert n % CH == 0
    nch = n // CH
    cpt = -(-nch // _NS)


    mesh = plsc.VectorSubcoreMesh(core_axis_name="c", subcore_axis_name="s")

    @functools.partial(
        pl.kernel,
        out_type=(
            jax.ShapeDtypeStruct((n, L), jnp.float32),
            jax.ShapeDtypeStruct((n, L), jnp.float32),
        ),
        mesh=mesh,
        scratch_types=dict(
            hbuf0=pltpu.VMEM((CE, L), jnp.float32),
            hbuf1=pltpu.VMEM((CE, L), jnp.float32),
            ibuf0=pltpu.VMEM((CE,), jnp.int32),
            ibuf1=pltpu.VMEM((CE,), jnp.int32),
            cbuf=pltpu.VMEM((CH, L), jnp.float32),
            accum=pltpu.VMEM_SHARED((n, L), jnp.float32),
            lsem0=pltpu.SemaphoreType.DMA,
            lsem1=pltpu.SemaphoreType.DMA,
            ssem0=pltpu.SemaphoreType.DMA,
            ssem1=pltpu.SemaphoreType.DMA,
        ),
        compiler_params=pltpu.CompilerParams(use_tc_tiling_on_sc=False),
    )
    def k(h_hbm, s_hbm, r_hbm, out_s, out_r,
          hbuf0, hbuf1, ibuf0, ibuf1, cbuf, accum,
          lsem0, lsem1, ssem0, ssem1):
        c = lax.axis_index("c")
        sid = lax.axis_index("s")
        g0 = sid * kpt
        g1 = jnp.minimum(chunks, g0 + kpt)
        n_my = jnp.maximum(0, g1 - g0)

        c0 = sid * cpt
        c1 = jnp.minimum(nch, c0 + cpt)

        bufs = ((hbuf0, ibuf0, lsem0, ssem0), (hbuf1, ibuf1, lsem1, ssem1))

        def run_pass(idx_hbm, out_hbm):
            def zfill(i, carry):
                cbuf[i, :] = jnp.zeros((L,), jnp.float32)
                return carry

            lax.fori_loop(0, CH, zfill, 0)

            def zbody(j, carry):
                pltpu.sync_copy(cbuf, accum.at[pl.ds(j * CH, CH)])
                return carry

            lax.fori_loop(c0, c1, zbody, 0)
            plsc.subcore_barrier()

            def loads(i, b):
                hb, ib, ls, _ = bufs[b]
                g = g0 + i
                pltpu.async_copy(h_hbm.at[pl.ds(g * CE, CE)], hb, ls)
                pltpu.async_copy(idx_hbm.at[pl.ds(g * CE, CE)], ib, ls)

            def drain_scatters(b):
                hb, ib, _, ss = bufs[b]
                pltpu.make_async_copy(hb, accum.at[ib], ss).wait()

            @pl.when(n_my > 0)
            def _():
                loads(0, 0)

            def process(i, b):
                hb, ib, ls, ss = bufs[b]

                @pl.when(i < n_my)
                def _():
                    g = g0 + i
                    pltpu.make_async_copy(h_hbm.at[pl.ds(g * CE, CE)], hb, ls).wait()
                    pltpu.make_async_copy(idx_hbm.at[pl.ds(g * CE, CE)], ib, ls).wait()

                    @pl.when(i + 1 < n_my)
                    def _():
                        @pl.when(i >= 1)
                        def _():
                            drain_scatters(1 - b)

                        loads(i + 1, 1 - b)

                    pltpu.async_copy(hb, accum.at[ib], ss, add=True)

            def pairbody(q, carry):
                process(2 * q, 0)
                process(2 * q + 1, 1)
                return carry

            lax.fori_loop(0, (kpt + 1) // 2, pairbody, 0)
            @pl.when(n_my >= 1)
            def _():
                drain_scatters((0))

            @pl.when(n_my >= 2)
            def _():
                drain_scatters((1))

            plsc.subcore_barrier()

            def wbody(j, carry):
                pltpu.sync_copy(accum.at[pl.ds(j * CH, CH)], cbuf)
                pltpu.sync_copy(cbuf, out_hbm.at[pl.ds(j * CH, CH)])
                return carry

            lax.fori_loop(c0, c1, wbody, 0)

        @pl.when(c == 0)
        def _():
            run_pass(s_hbm, out_s)

        @pl.when(c == 1)
        def _():
            run_pass(r_hbm, out_r)

    return k(h_e, senders, receivers)


def _node_pipeline(nodes, sentP, recvP, globals_, params):
    N, F = nodes.shape
    L = 16
    P8 = 128 // L
    (eW1, eb1), (eW2, eb2) = params["enc_node"]
    (pW1, pb1), (pW2, pb2) = params["proc_node"]
    (dW1, db1), (dW2, db2), (dW3, db3) = params["dec_node"]
    pA = pW1[:L]
    pB = pW1[L : 2 * L]
    pC = pW1[2 * L : 3 * L]
    pD = pW1[3 * L :]
    g_row = globals_[None, :]
    (_, _), (eW2e, eb2e) = params["enc_edge"]
    eyeP = jnp.eye(P8, dtype=jnp.float32)
    BDB = jnp.kron(eyeP, eW2e @ pB)
    BDC = jnp.kron(eyeP, eW2e @ pC)

    B = 2048
    grid = -(-N // B)

    def body(n_ref, s_ref, r_ref, g_ref,
             ew1, eb1_, ew2, eb2_, pa, bdb, bdc, pd, pb1_, pw2, pb2_,
             dw1, db1_, dw2, db2_, dw3t, db3_, o_ref):
        x = n_ref[...]
        xT = jnp.transpose(x)
        hn = _softplus(jnp.dot(x, ew1[...], preferred_element_type=jnp.float32) + eb1_[...])
        hn = jnp.dot(hn, ew2[...], preferred_element_type=jnp.float32) + eb2_[...]
        cP = (jnp.dot(s_ref[...], bdb[...], preferred_element_type=jnp.float32)
              + jnp.dot(r_ref[...], bdc[...], preferred_element_type=jnp.float32))
        c = jnp.stack([cP[:, L * e : L * (e + 1)] for e in range(P8)], axis=1)
        c = c.reshape(B, L)
        gv = jnp.dot(g_ref[...], pd[...], preferred_element_type=jnp.float32) + pb1_[...]
        t = jnp.dot(hn, pa[...], preferred_element_type=jnp.float32) + c + gv
        h = jnp.dot(_softplus(t), pw2[...], preferred_element_type=jnp.float32) + pb2_[...]
        d = _softplus(jnp.dot(h, dw1[...], preferred_element_type=jnp.float32) + db1_[...])
        d = _softplus(jnp.dot(d, dw2[...], preferred_element_type=jnp.float32) + db2_[...])
        accT = (jnp.dot(dw3t[...], jnp.transpose(d),
                        preferred_element_type=jnp.float32)
                + db3_[...])
        nvT = xT[F - 1 : F] + accT * DT
        npT = xT[0:1] + nvT * DT
        o_ref[...] = jnp.concatenate([npT, xT[2:F], nvT, accT], axis=0)

    full = lambda a: pl.BlockSpec(a.shape, lambda i: tuple(0 for _ in a.shape))
    row2 = lambda v: v[None, :]
    weights = [eW1, row2(eb1), eW2, row2(eb2),
               pA, BDB, BDC, pD, row2(pb1), pW2, row2(pb2),
               dW1, row2(db1), dW2, row2(db2), dW3.T, db3.reshape(1, 1)]
    out = pl.pallas_call(
        body,
        grid=(grid,),
        in_specs=[
            pl.BlockSpec((B, F), lambda i: (i, 0)),
            pl.BlockSpec((B // P8, 128), lambda i: (i, 0)),
            pl.BlockSpec((B // P8, 128), lambda i: (i, 0)),
            full(g_row),
        ] + [full(w) for w in weights],
        out_specs=pl.BlockSpec((F + 1, B), lambda i: (0, i)),
        out_shape=jax.ShapeDtypeStruct((F + 1, N), jnp.float32),
    )(nodes, sentP, recvP, g_row, *weights)
    return out


def kernel(nodes, edges, senders, receivers, globals_, params):
    N = nodes.shape[0]
    h_e = _edge_encoder(edges, params["enc_edge"])
    sent, recv = _sc_dual_segment_sum(h_e, senders, receivers, N)
    outT = _node_pipeline(
        nodes, sent.reshape(N // 8, 128), recv.reshape(N // 8, 128),
        globals_, params)
    next_nodes = outT.T
    next_pos = outT[0]
    next_edges = (next_pos[1:] - next_pos[:-1]).reshape(-1, 1)
    new_globals = jnp.concatenate((globals_[:1] + 1.0, globals_[1:]))
    return next_nodes, next_edges, new_globals
